# Initial kernel scaffold; baseline (speedup 1.0000x reference)
import jax
import jax.numpy as jnp
from jax.experimental import pallas as pl


def kernel(x, h, edge_index, si_Wl, si_bl, si_Wr, sh_Wl, sh_bl, sh_Wr, ssx_Wl, ssx_bl, ssx_Wr, ssh_Wl, ssh_bl, ssh_Wr, sux_Wl, sux_bl, sux_Wr, suh_Wl, suh_bl, suh_Wr, scx_Wl, scx_bl, scx_Wr, sch_Wl, sch_bl, sch_Wr, Wg1, bg1, Wg2, bg2, Wb1, bb1, Wb2, bb2):
    raise NotImplementedError("write your pallas kernel here")


# speedup vs baseline: 5.5471x; 5.5471x over previous
import functools

import jax
import jax.numpy as jnp
from jax import lax
from jax.experimental import pallas as pl
from jax.experimental.pallas import tpu as pltpu
from jax.experimental.pallas import tpu_sc as plsc

_NC = 2
_NS = 16
_NW = _NC * _NS
_CH = 128


def _make_segsum(n_rows, n_pad, e_pad):
    n_chunks = e_pad // _CH
    cpw = n_chunks // _NW
    zr = n_pad // _NS
    G = 16
    ng = cpw // G
    mesh = plsc.VectorSubcoreMesh(core_axis_name="c", subcore_axis_name="s")

    @functools.partial(
        pl.kernel, mesh=mesh,
        out_type=jax.ShapeDtypeStruct((_NC, n_pad, 128), jnp.float32),
        scratch_types=[
            pltpu.VMEM((G, _CH), jnp.int32),
            pltpu.VMEM((G, _CH), jnp.int32),
            pltpu.VMEM((_CH, 128), jnp.float32),
            pltpu.VMEM_SHARED((n_pad, 128), jnp.float32),
            pltpu.SemaphoreType.DMA,
        ],
    )
    def seg(table, src2, dst2, zrow, out, src_v, dst_v, rows_v, acc_sh, sem):
        cid = lax.axis_index("c")
        sid = lax.axis_index("s")
        wid = sid * _NC + cid
        pltpu.sync_copy(zrow, acc_sh.at[pl.ds(sid * zr, zr)])
        base = wid * cpw
        plsc.subcore_barrier()

        def group(g, carry):
            pltpu.sync_copy(src2.at[pl.ds(base + g * G, G)], src_v)
            pltpu.sync_copy(dst2.at[pl.ds(base + g * G, G)], dst_v)

            def chunk(j, c2):
                pltpu.async_copy(table.at[src_v.at[j]], rows_v, sem).wait()
                pltpu.sync_copy(rows_v, acc_sh.at[dst_v.at[j]], add=True)
                return c2

            lax.fori_loop(0, G, chunk, carry)
            return carry

        lax.fori_loop(0, ng, group, 0)
        plsc.subcore_barrier()
        pltpu.sync_copy(acc_sh.at[pl.ds(sid * zr, zr)],
                        out.at[cid, pl.ds(sid * zr, zr)])

    return seg


def _make_deg(n_pad, e_pad):
    n_chunks = e_pad // _CH
    cpw = n_chunks // _NW
    zr = n_pad // _NS
    G = 16
    ng = cpw // G
    mesh = plsc.VectorSubcoreMesh(core_axis_name="c", subcore_axis_name="s")

    @functools.partial(
        pl.kernel, mesh=mesh,
        out_type=jax.ShapeDtypeStruct((_NC, n_pad, 128), jnp.float32),
        scratch_types=[
            pltpu.VMEM((G, _CH), jnp.int32),
            pltpu.VMEM((_CH, 128), jnp.float32),
            pltpu.VMEM_SHARED((n_pad, 128), jnp.float32),
        ],
    )
    def degk(dst2, zrow, ones_hbm, out, dst_v, ones_v, acc_sh):
        cid = lax.axis_index("c")
        sid = lax.axis_index("s")
        wid = sid * _NC + cid
        pltpu.sync_copy(zrow, acc_sh.at[pl.ds(sid * zr, zr)])
        pltpu.sync_copy(ones_hbm, ones_v)
        base = wid * cpw
        plsc.subcore_barrier()

        def group(g, carry):
            pltpu.sync_copy(dst2.at[pl.ds(base + g * G, G)], dst_v)

            def chunk(j, c2):
                pltpu.sync_copy(ones_v, acc_sh.at[dst_v.at[j]], add=True)
                return c2

            lax.fori_loop(0, G, chunk, carry)
            return carry

        lax.fori_loop(0, ng, group, 0)
        plsc.subcore_barrier()
        pltpu.sync_copy(acc_sh.at[pl.ds(sid * zr, zr)],
                        out.at[cid, pl.ds(sid * zr, zr)])

    return degk


def _dotT(a, w):
    return lax.dot_general(a, w, (((1,), (1,)), ((), ())),
                           preferred_element_type=jnp.float32)


def _row_specs(nb, B):
    full = pl.BlockSpec((B, 128), lambda i: (i, 0))
    w = pl.BlockSpec((128, 128), lambda i: (0, 0))
    b = pl.BlockSpec((1, 128), lambda i: (0, 0))
    s0 = pl.BlockSpec((1, B, 128), lambda i: (0, i, 0))
    s1 = pl.BlockSpec((1, B, 128), lambda i: (1, i, 0))
    d0 = pl.BlockSpec((1, B, 16), lambda i: (0, i, 0))
    d1 = pl.BlockSpec((1, B, 16), lambda i: (1, i, 0))
    d = pl.BlockSpec((B, 8), lambda i: (i, 0))
    return full, w, b, s0, s1, d0, d1, d


def _tc1(h, sumh, deg8, sh_Wl, sh_bl, sh_Wr, Wb1, bb1, Wb2, bb2, B=1000):
    n = h.shape[0]
    nb = n // B
    full, w, b, s0, s1, d0, d1, d = _row_specs(nb, B)

    def body(h_r, sh0_r, sh1_r, dg0_r, dg1_r, wl_r, bl_r, wr_r,
             wb1_r, bb1_r, wb2_r, bb2_r, hn_r, invd_r):
        deg = dg0_r[0] + dg1_r[0]
        invd = 1.0 / jnp.maximum(deg, 1.0)
        mean_h = (sh0_r[0] + sh1_r[0]) * invd[:, :1]
        hv = h_r[...]
        hn0 = _dotT(mean_h, wl_r[...]) + bl_r[...] + _dotT(hv, wr_r[...])
        beta = jnp.tanh(_dotT(hv, wb1_r[...]) + bb1_r[...]
                        + _dotT(hn0, wb2_r[...]) + bb2_r[...])
        hn_r[...] = hv + beta
        invd_r[...] = invd[:, :8]

    return pl.pallas_call(
        body,
        grid=(nb,),
        in_specs=[full, s0, s1, s0, s1, w, b, w, w, b, w, b],
        out_specs=[full, d],
        out_shape=[jax.ShapeDtypeStruct((n, 128), jnp.float32),
                   jax.ShapeDtypeStruct((n, 8), jnp.float32)],
    )(h, sumh, sumh, deg8, deg8, sh_Wl, sh_bl, sh_Wr, Wb1, bb1, Wb2, bb2)


def _tc2(x, hN, sumx, sumhn, invd8,
         ssx_Wl, ssx_bl, ssx_Wr, ssh_Wl, ssh_bl, ssh_Wr,
         sux_Wl, sux_bl, sux_Wr, suh_Wl, suh_bl, suh_Wr, B=1000):
    n = x.shape[0]
    nb = n // B
    full, w, b, s0, s1, d0, d1, d = _row_specs(nb, B)

    def body(x_r, hn_r, sx0_r, sx1_r, shn0_r, shn1_r, invd_r,
             axl_r, axb_r, axr_r, ahl_r, ahb_r, ahr_r,
             uxl_r, uxb_r, uxr_r, uhl_r, uhb_r, uhr_r,
             q_r, z_r, mx_r):
        invd = invd_r[:, :1]
        mx = (sx0_r[0] + sx1_r[0]) * invd
        mhn = (shn0_r[0] + shn1_r[0]) * invd
        xv = x_r[...]
        hnv = hn_r[...]
        r = jax.nn.sigmoid(_dotT(mx, axl_r[...]) + axb_r[...] + _dotT(xv, axr_r[...])
                           + _dotT(mhn, ahl_r[...]) + ahb_r[...] + _dotT(hnv, ahr_r[...]))
        z = jax.nn.sigmoid(_dotT(mx, uxl_r[...]) + uxb_r[...] + _dotT(xv, uxr_r[...])
                           + _dotT(mhn, uhl_r[...]) + uhb_r[...] + _dotT(hnv, uhr_r[...]))
        q_r[...] = r * hnv
        z_r[...] = z
        mx_r[...] = mx

    return pl.pallas_call(
        body,
        grid=(nb,),
        in_specs=[full, full, s0, s1, s0, s1, d,
                  w, b, w, w, b, w, w, b, w, w, b, w],
        out_specs=[full, full, full],
        out_shape=[jax.ShapeDtypeStruct((n, 128), jnp.float32)] * 3,
    )(x, hN, sumx, sumx, sumhn, sumhn, invd8,
      ssx_Wl, ssx_bl, ssx_Wr, ssh_Wl, ssh_bl, ssh_Wr,
      sux_Wl, sux_bl, sux_Wr, suh_Wl, suh_bl, suh_Wr)


def _tc3(x, hN, q, z, mx, sumq, invd8,
         scx_Wl, scx_bl, scx_Wr, sch_Wl, sch_bl, sch_Wr, B=1000):
    n = x.shape[0]
    nb = n // B
    full, w, b, s0, s1, d0, d1, d = _row_specs(nb, B)

    def body(x_r, hn_r, q_r, z_r, mx_r, sq0_r, sq1_r, invd_r,
             cxl_r, cxb_r, cxr_r, chl_r, chb_r, chr_r, out_r):
        invd = invd_r[:, :1]
        mq = (sq0_r[0] + sq1_r[0]) * invd
        ht = jnp.tanh(_dotT(mx_r[...], cxl_r[...]) + cxb_r[...]
                      + _dotT(x_r[...], cxr_r[...])
                      + _dotT(mq, chl_r[...]) + chb_r[...]
                      + _dotT(q_r[...], chr_r[...]))
        zv = z_r[...]
        out_r[...] = (1.0 - zv) * hn_r[...] + zv * ht

    return pl.pallas_call(
        body,
        grid=(nb,),
        in_specs=[full, full, full, full, full, s0, s1, d,
                  w, b, w, w, b, w],
        out_specs=full,
        out_shape=jax.ShapeDtypeStruct((n, 128), jnp.float32),
    )(x, hN, q, z, mx, sumq, sumq, invd8,
      scx_Wl, scx_bl, scx_Wr, sch_Wl, sch_bl, sch_Wr)


def kernel(x, h, edge_index,
           si_Wl, si_bl, si_Wr, sh_Wl, sh_bl, sh_Wr,
           ssx_Wl, ssx_bl, ssx_Wr, ssh_Wl, ssh_bl, ssh_Wr,
           sux_Wl, sux_bl, sux_Wr, suh_Wl, suh_bl, suh_Wr,
           scx_Wl, scx_bl, scx_Wr, sch_Wl, sch_bl, sch_Wr,
           Wg1, bg1, Wg2, bg2, Wb1, bb1, Wb2, bb2):
    n = x.shape[0]
    e = edge_index.shape[1]
    ealign = _CH * _NW * 8
    ep = ((e + ealign - 1) // ealign) * ealign
    npad = ((n + 16 + _NS * 8 - 1) // (_NS * 8)) * (_NS * 8)
    pad = ep - e

    src = edge_index[0]
    dst = edge_index[1]
    if pad:
        ar = jnp.arange(pad, dtype=jnp.int32)
        src = jnp.concatenate([src, ar % n])
        dst = jnp.concatenate([dst, n + (ar % 16)])
    src2 = src.reshape(-1, _CH)
    dst2 = dst.reshape(-1, _CH)

    zrow = jnp.zeros((npad // _NS, 128), jnp.float32)
    ones128 = jnp.ones((_CH, 128), jnp.float32)

    seg = _make_segsum(n, npad, ep)
    degk = _make_deg(npad, ep)

    deg128 = degk(dst2, zrow, ones128)
    sumh = seg(h, src2, dst2, zrow)
    hN, invd8 = _tc1(h, sumh, deg128, sh_Wl, sh_bl.reshape(1, -1), sh_Wr,
                     Wb1, bb1.reshape(1, -1), Wb2, bb2.reshape(1, -1))
    sumx = seg(x, src2, dst2, zrow)
    sumhn = seg(hN, src2, dst2, zrow)
    q, z, mx = _tc2(x, hN, sumx, sumhn, invd8,
                    ssx_Wl, ssx_bl.reshape(1, -1), ssx_Wr,
                    ssh_Wl, ssh_bl.reshape(1, -1), ssh_Wr,
                    sux_Wl, sux_bl.reshape(1, -1), sux_Wr,
                    suh_Wl, suh_bl.reshape(1, -1), suh_Wr)
    sumq = seg(q, src2, dst2, zrow)
    h_new = _tc3(x, hN, q, z, mx, sumq, invd8,
                 scx_Wl, scx_bl.reshape(1, -1), scx_Wr,
                 sch_Wl, sch_bl.reshape(1, -1), sch_Wr)
    return h_new
